# baseline (speedup 1.0000x reference)
import functools

import jax
import jax.numpy as jnp
from jax import lax
from jax.experimental import pallas as pl
from jax.experimental.pallas import tpu as pltpu
from jax.experimental.pallas import tpu_sc as plsc

B = 65536
K = 64
BR = 2048
NB = B // BR

_info = plsc.get_sparse_core_info()
_NC, _NS = _info.num_cores, _info.num_subcores
NW = _NC * _NS
BPW = B // NW


def _sc_gather(query_preds, seg_starts, seg_lens):
    mesh = plsc.VectorSubcoreMesh(core_axis_name="c", subcore_axis_name="s")

    @functools.partial(
        pl.kernel,
        mesh=mesh,
        out_type=[
            jax.ShapeDtypeStruct((B,), jnp.int32),
            jax.ShapeDtypeStruct((B,), jnp.int32),
        ],
        scratch_types=[
            pltpu.VMEM((BPW,), jnp.int32),
            pltpu.VMEM((BPW,), jnp.int32),
            pltpu.VMEM((BPW,), jnp.int32),
            pltpu.SemaphoreType.DMA,
            pltpu.SemaphoreType.DMA,
        ],
    )
    def body(qp_hbm, starts_hbm, lens_hbm, out_s_hbm, out_l_hbm,
             qp_v, s_v, l_v, sem_s, sem_l):
        wid = lax.axis_index("s") * _NC + lax.axis_index("c")
        base = wid * BPW
        pltpu.sync_copy(qp_hbm.at[pl.ds(base, BPW)], qp_v)
        cp_s = pltpu.async_copy(starts_hbm.at[qp_v], s_v, sem_s)
        cp_l = pltpu.async_copy(lens_hbm.at[qp_v], l_v, sem_l)
        cp_s.wait()
        cp_l.wait()
        pltpu.sync_copy(s_v, out_s_hbm.at[pl.ds(base, BPW)])
        pltpu.sync_copy(l_v, out_l_hbm.at[pl.ds(base, BPW)])

    return body(query_preds, seg_starts, seg_lens)


BR2 = BR // 2


BR2 = BR // 2


def _tc_expand_body(se_ref, so_ref, le_ref, lo_ref, offs_ref,
                    item_ref, mask_ref, qidx_ref):
    i = pl.program_id(0)
    se_c = jnp.reshape(se_ref[0, 0, :], (BR2, 1))
    so_c = jnp.reshape(so_ref[0, 0, :], (BR2, 1))
    le_c = jnp.reshape(le_ref[0, 0, :], (BR2, 1))
    lo_c = jnp.reshape(lo_ref[0, 0, :], (BR2, 1))
    o = offs_ref[0:1, :]
    s2 = jnp.concatenate(
        [jnp.broadcast_to(se_c, (BR2, K)), jnp.broadcast_to(so_c, (BR2, K))],
        axis=1)
    l2 = jnp.concatenate(
        [jnp.broadcast_to(le_c, (BR2, K)), jnp.broadcast_to(lo_c, (BR2, K))],
        axis=1)
    o2 = jnp.concatenate([o, o], axis=1)
    item_ref[...] = s2 + o2
    mask_ref[...] = o2 < l2
    r = lax.broadcasted_iota(jnp.int32, (BR2, 2 * K), 0)
    c_hi = lax.broadcasted_iota(jnp.int32, (BR2, 2 * K), 1) // K
    qidx_ref[...] = i * BR + r * 2 + c_hi


def _tc_expand(starts_d, lens_d, offs):
    grid = (NB,)
    return pl.pallas_call(
        _tc_expand_body,
        grid=grid,
        in_specs=[
            pl.BlockSpec((1, 1, BR2), lambda i: (i, 0, 0)),
            pl.BlockSpec((1, 1, BR2), lambda i: (NB + i, 0, 0)),
            pl.BlockSpec((1, 1, BR2), lambda i: (i, 0, 0)),
            pl.BlockSpec((1, 1, BR2), lambda i: (NB + i, 0, 0)),
            pl.BlockSpec((8, K), lambda i: (0, 0)),
        ],
        out_specs=[
            pl.BlockSpec((BR2, 2 * K), lambda i: (i, 0)),
            pl.BlockSpec((BR2, 2 * K), lambda i: (i, 0)),
            pl.BlockSpec((BR2, 2 * K), lambda i: (i, 0)),
        ],
        out_shape=[
            jax.ShapeDtypeStruct((B // 2, 2 * K), jnp.int32),
            jax.ShapeDtypeStruct((B // 2, 2 * K), jnp.bool_),
            jax.ShapeDtypeStruct((B // 2, 2 * K), jnp.int32),
        ],
    )(starts_d, starts_d, lens_d, lens_d, offs)


def kernel(query_preds, max_pairs, seg_starts, seg_lens):
    qp_d = jnp.concatenate([query_preds[0::2], query_preds[1::2]])
    starts_d, lens_d = _sc_gather(qp_d, seg_starts, seg_lens)
    pad = (jnp.asarray(max_pairs, jnp.int32) - K)
    offs = jnp.arange(K, dtype=jnp.int32) + pad
    offs_b = jnp.broadcast_to(offs[None, :], (8, K))
    item2, mask2, qidx2 = _tc_expand(
        starts_d.reshape(2 * NB, 1, BR2), lens_d.reshape(2 * NB, 1, BR2),
        offs_b)
    return (item2.reshape(B, K), mask2.reshape(B, K), qidx2.reshape(B, K))
